# baseline (speedup 1.0000x reference)
import functools

import jax
import jax.numpy as jnp
from jax import lax
from jax.experimental import pallas as pl
from jax.experimental.pallas import tpu as pltpu
from jax.experimental.pallas import tpu_sc as plsc

N_NODES = 10000
N_EDGES = 160000
DIM = 256
N_REL = 8
LANES = 16

N_SEG = N_NODES * N_REL
ACC_ROWS = N_SEG + 8
N_CHUNK = 18
CHUNKS_PER_CORE = N_CHUNK // 2
EDGE_B = 128
N_SUBCORES = 16
NBLK = 81
EPT = NBLK * EDGE_B
E_PAD = N_SUBCORES * EPT
ZROWS = 500
TABLE_ROWS = N_CHUNK * N_NODES + ZROWS


def _sc_body(table_hbm, src_hbm, seg_hbm, out_hbm,
             src_v, seg_v, idx, rows, zbuf, obuf, acc, gsem, ssem):
    core = lax.axis_index("c")
    tile = lax.axis_index("s")

    pltpu.sync_copy(src_hbm.at[pl.ds(tile * EPT, EPT)], src_v)
    pltpu.sync_copy(seg_hbm.at[tile], seg_v)
    pltpu.sync_copy(table_hbm.at[pl.ds(N_CHUNK * N_NODES, ZROWS)], zbuf)

    for k in range(CHUNKS_PER_CORE):
        chunk = core * CHUNKS_PER_CORE + k
        cbase = chunk * N_NODES

        for q in range(10):
            pltpu.sync_copy(zbuf, acc.at[pl.ds(tile * 5000 + q * ZROWS, ZROWS)])
        plsc.subcore_barrier()

        def stage(b, K):
            for j in range(EDGE_B // LANES):
                idx[K][pl.ds(j * LANES, LANES)] = (
                    src_v[pl.ds(b * EDGE_B + j * LANES, LANES)] + cbase)

        def fire_g(b, K, S):
            stage(b, K)
            pltpu.async_copy(table_hbm.at[idx[K]], rows[K], gsem[S])

        def drain_g(K, S):
            pltpu.make_async_copy(table_hbm.at[idx[K]], rows[K],
                                  gsem[S]).wait()

        def fire_s(b, K, S):
            pltpu.async_copy(rows[K], acc.at[seg_v.at[b]], ssem[S], add=True)

        def drain_s(b, K, S):
            pltpu.make_async_copy(rows[K], acc.at[seg_v.at[b]],
                                  ssem[S]).wait()

        def group(g, S, first, last):
            T = 1 - S
            for u in range(3):
                drain_g(3 * S + u, S)
            for u in range(3):
                fire_s(3 * g + u, 3 * S + u, S)
            if not first:
                for u in range(3):
                    drain_s(3 * (g - 1) + u, 3 * T + u, T)
            if not last:
                for u in range(3):
                    fire_g(3 * (g + 1) + u, 3 * T + u, T)

        for u in range(3):
            fire_g(u, u, 0)
        group(0, 0, True, False)
        group(1, 1, False, False)

        def body(p, carry):
            group(2 + 2 * p, 0, False, False)
            group(3 + 2 * p, 1, False, False)
            return carry

        lax.fori_loop(0, (NBLK // 3 - 3) // 2, body, 0)
        group(NBLK // 3 - 1, 0, False, True)
        for u in range(3):
            drain_s(NBLK - 3 + u, u, 0)
        plsc.subcore_barrier()

        for q in range(10):
            r0 = tile * 5000 + q * ZROWS
            pltpu.sync_copy(acc.at[pl.ds(r0, ZROWS)], obuf)
            pltpu.sync_copy(
                obuf,
                out_hbm.at[pl.ds(r0, ZROWS), pl.ds(chunk * LANES, LANES)])


@functools.lru_cache(maxsize=1)
def _build_sc_kernel():
    mesh = plsc.VectorSubcoreMesh(core_axis_name="c", subcore_axis_name="s")
    return pl.kernel(
        _sc_body,
        mesh=mesh,
        out_type=jax.ShapeDtypeStruct((N_SEG, N_CHUNK * LANES), jnp.float32),
        scratch_types=[
            pltpu.VMEM((EPT,), jnp.int32),
            pltpu.VMEM((NBLK, EDGE_B), jnp.int32),
            [pltpu.VMEM((EDGE_B,), jnp.int32)] * 6,
            [pltpu.VMEM((EDGE_B, LANES), jnp.float32)] * 6,
            pltpu.VMEM((ZROWS, LANES), jnp.float32),
            pltpu.VMEM((ZROWS, LANES), jnp.float32),
            pltpu.VMEM_SHARED((ACC_ROWS, LANES), jnp.float32),
            [pltpu.SemaphoreType.DMA] * 2,
            [pltpu.SemaphoreType.DMA] * 2,
        ],
        compiler_params=pltpu.CompilerParams(use_tc_tiling_on_sc=False),
    )


def _sc_segment_sum(table, src, seg3):
    return _build_sc_kernel()(table, src, seg3)


def _dense_layer(xin, araw, wcat, b2d, relu):
    bn = 400
    grid = (N_NODES // bn,)

    def body(x_ref, a_ref, w_ref, b_ref, o_ref):
        x = x_ref[...]
        acc = jnp.dot(x, w_ref[0:DIM, :],
                      preferred_element_type=jnp.float32) + b_ref[...]
        for r in range(N_REL):
            blk = a_ref[:, r * 288:(r + 1) * 288]
            feat = blk[:, :DIM]
            cnt = blk[:, DIM:DIM + 1]
            inv = 1.0 / jnp.maximum(cnt, 1.0)
            w_r = w_ref[DIM + r * DIM:DIM + (r + 1) * DIM, :]
            acc = acc + jnp.dot(feat * inv, w_r,
                                preferred_element_type=jnp.float32)
        o_ref[...] = jnp.maximum(acc, 0.0) if relu else acc

    return pl.pallas_call(
        body,
        grid=grid,
        in_specs=[
            pl.BlockSpec((bn, DIM), lambda i: (i, 0)),
            pl.BlockSpec((bn, N_REL * 288), lambda i: (i, 0)),
            pl.BlockSpec((DIM + N_REL * DIM, DIM), lambda i: (0, 0)),
            pl.BlockSpec((1, DIM), lambda i: (0, 0)),
        ],
        out_specs=pl.BlockSpec((bn, DIM), lambda i: (i, 0)),
        out_shape=jax.ShapeDtypeStruct((N_NODES, DIM), jnp.float32),
    )(xin, araw, wcat, b2d)


def _rgcn_layer(xin, src, seg3, W, root, b, relu):
    xcols = xin.reshape(N_NODES, LANES, LANES).transpose(1, 0, 2)
    xcols = xcols.reshape(LANES * N_NODES, LANES)
    ones = jnp.ones((2 * N_NODES, LANES), jnp.float32)
    zeros = jnp.zeros((ZROWS, LANES), jnp.float32)
    table = jnp.concatenate([xcols, ones, zeros], axis=0)

    araw = _sc_segment_sum(table, src, seg3)
    araw = araw.reshape(N_NODES, N_REL * 288)

    wcat = jnp.concatenate(
        [root, W.reshape(N_REL * DIM, DIM)], axis=0)
    return _dense_layer(xin, araw, wcat, b.reshape(1, DIM), relu)


def kernel(x, edge_index, edge_type, W1, root1, b1, W2, root2, b2):
    src = edge_index[0]
    dst = edge_index[1]
    seg = dst * N_REL + edge_type

    pad = E_PAD - N_EDGES
    src_p = jnp.concatenate([src, jnp.zeros((pad,), jnp.int32)])
    seg_p = jnp.concatenate([seg, jnp.full((pad,), N_SEG, jnp.int32)])
    seg3 = seg_p.reshape(N_SUBCORES, NBLK, EDGE_B)

    h1 = _rgcn_layer(x, src_p, seg3, W1, root1, b1, relu=True)
    h2 = _rgcn_layer(h1, src_p, seg3, W2, root2, b2, relu=False)
    return h2


# speedup vs baseline: 1.0376x; 1.0376x over previous
import functools

import jax
import jax.numpy as jnp
from jax import lax
from jax.experimental import pallas as pl
from jax.experimental.pallas import tpu as pltpu
from jax.experimental.pallas import tpu_sc as plsc

N_NODES = 10000
N_EDGES = 160000
DIM = 256
N_REL = 8
LANES = 16

N_SEG = N_NODES * N_REL
ACC_ROWS = N_SEG + 8
N_CHUNK = 18
CHUNKS_PER_CORE = N_CHUNK // 2
EDGE_B = 128
N_SUBCORES = 16
NBLK = 81
EPT = NBLK * EDGE_B
E_PAD = N_SUBCORES * EPT
ZROWS = 500
TABLE_ROWS = N_CHUNK * N_NODES + ZROWS


def _sc_body(table_hbm, src_hbm, seg_hbm, out_hbm,
             src_v, seg_v, idx, rows, zbuf, obuf, acc, gsem, ssem):
    core = lax.axis_index("c")
    tile = lax.axis_index("s")

    pltpu.sync_copy(src_hbm.at[pl.ds(tile * EPT, EPT)], src_v)
    pltpu.sync_copy(seg_hbm.at[tile], seg_v)
    pltpu.sync_copy(table_hbm.at[pl.ds(N_CHUNK * N_NODES, ZROWS)], zbuf)

    for k in range(CHUNKS_PER_CORE):
        chunk = core * CHUNKS_PER_CORE + k
        cbase = chunk * N_NODES

        for q in range(10):
            pltpu.sync_copy(zbuf, acc.at[pl.ds(tile * 5000 + q * ZROWS, ZROWS)])
        plsc.subcore_barrier()

        def stage(b, K):
            for j in range(EDGE_B // LANES):
                idx[K][pl.ds(j * LANES, LANES)] = (
                    src_v[pl.ds(b * EDGE_B + j * LANES, LANES)] + cbase)

        def fire_g(b, K):
            stage(b, K)
            pltpu.async_copy(table_hbm.at[idx[K]], rows[K], gsem[K])

        def drain_g(K):
            pltpu.make_async_copy(table_hbm.at[idx[K]], rows[K],
                                  gsem[K]).wait()

        def scat(b, K):
            pltpu.sync_copy(rows[K], acc.at[seg_v.at[b]], add=True)

        fire_g(0, 0)

        def pair(p, carry):
            b0 = 2 * p
            fire_g(b0 + 1, 1)
            drain_g(0)
            scat(b0, 0)
            fire_g(b0 + 2, 0)
            drain_g(1)
            scat(b0 + 1, 1)
            return carry

        lax.fori_loop(0, (NBLK - 1) // 2, pair, 0)
        drain_g(0)
        scat(NBLK - 1, 0)
        plsc.subcore_barrier()

        for q in range(10):
            r0 = tile * 5000 + q * ZROWS
            pltpu.sync_copy(acc.at[pl.ds(r0, ZROWS)], obuf)
            pltpu.sync_copy(
                obuf,
                out_hbm.at[pl.ds(r0, ZROWS), pl.ds(chunk * LANES, LANES)])


@functools.lru_cache(maxsize=1)
def _build_sc_kernel():
    mesh = plsc.VectorSubcoreMesh(core_axis_name="c", subcore_axis_name="s")
    return pl.kernel(
        _sc_body,
        mesh=mesh,
        out_type=jax.ShapeDtypeStruct((N_SEG, N_CHUNK * LANES), jnp.float32),
        scratch_types=[
            pltpu.VMEM((EPT,), jnp.int32),
            pltpu.VMEM((NBLK, EDGE_B), jnp.int32),
            [pltpu.VMEM((EDGE_B,), jnp.int32)] * 3,
            [pltpu.VMEM((EDGE_B, LANES), jnp.float32)] * 3,
            pltpu.VMEM((ZROWS, LANES), jnp.float32),
            pltpu.VMEM((ZROWS, LANES), jnp.float32),
            pltpu.VMEM_SHARED((ACC_ROWS, LANES), jnp.float32),
            [pltpu.SemaphoreType.DMA] * 3,
            [pltpu.SemaphoreType.DMA] * 3,
        ],
        compiler_params=pltpu.CompilerParams(use_tc_tiling_on_sc=False),
    )


def _sc_segment_sum(table, src, seg3):
    return _build_sc_kernel()(table, src, seg3)


def _dense_layer(xin, araw, wcat, b2d, relu):
    bn = 400
    grid = (N_NODES // bn,)

    def body(x_ref, a_ref, w_ref, b_ref, o_ref):
        x = x_ref[...]
        acc = jnp.dot(x, w_ref[0:DIM, :],
                      preferred_element_type=jnp.float32) + b_ref[...]
        for r in range(N_REL):
            blk = a_ref[:, r * 288:(r + 1) * 288]
            feat = blk[:, :DIM]
            cnt = blk[:, DIM:DIM + 1]
            inv = 1.0 / jnp.maximum(cnt, 1.0)
            w_r = w_ref[DIM + r * DIM:DIM + (r + 1) * DIM, :]
            acc = acc + jnp.dot(feat * inv, w_r,
                                preferred_element_type=jnp.float32)
        o_ref[...] = jnp.maximum(acc, 0.0) if relu else acc

    return pl.pallas_call(
        body,
        grid=grid,
        in_specs=[
            pl.BlockSpec((bn, DIM), lambda i: (i, 0)),
            pl.BlockSpec((bn, N_REL * 288), lambda i: (i, 0)),
            pl.BlockSpec((DIM + N_REL * DIM, DIM), lambda i: (0, 0)),
            pl.BlockSpec((1, DIM), lambda i: (0, 0)),
        ],
        out_specs=pl.BlockSpec((bn, DIM), lambda i: (i, 0)),
        out_shape=jax.ShapeDtypeStruct((N_NODES, DIM), jnp.float32),
    )(xin, araw, wcat, b2d)


def _rgcn_layer(xin, src, seg3, W, root, b, relu):
    xcols = xin.reshape(N_NODES, LANES, LANES).transpose(1, 0, 2)
    xcols = xcols.reshape(LANES * N_NODES, LANES)
    ones = jnp.ones((2 * N_NODES, LANES), jnp.float32)
    zeros = jnp.zeros((ZROWS, LANES), jnp.float32)
    table = jnp.concatenate([xcols, ones, zeros], axis=0)

    araw = _sc_segment_sum(table, src, seg3)
    araw = araw.reshape(N_NODES, N_REL * 288)

    wcat = jnp.concatenate(
        [root, W.reshape(N_REL * DIM, DIM)], axis=0)
    return _dense_layer(xin, araw, wcat, b.reshape(1, DIM), relu)


def kernel(x, edge_index, edge_type, W1, root1, b1, W2, root2, b2):
    src = edge_index[0]
    dst = edge_index[1]
    seg = dst * N_REL + edge_type

    pad = E_PAD - N_EDGES
    src_p = jnp.concatenate([src, jnp.zeros((pad,), jnp.int32)])
    seg_p = jnp.concatenate([seg, jnp.full((pad,), N_SEG, jnp.int32)])
    seg3 = seg_p.reshape(N_SUBCORES, NBLK, EDGE_B)

    h1 = _rgcn_layer(x, src_p, seg3, W1, root1, b1, relu=True)
    h2 = _rgcn_layer(h1, src_p, seg3, W2, root2, b2, relu=False)
    return h2
